# Initial kernel scaffold; baseline (speedup 1.0000x reference)
import jax
import jax.numpy as jnp
from jax.experimental import pallas as pl


def kernel(a, codebook):
    raise NotImplementedError("write your pallas kernel here")


# speedup vs baseline: 1.2092x; 1.2092x over previous
import functools

import jax
import jax.numpy as jnp
from jax import lax
from jax.experimental import pallas as pl
from jax.experimental.pallas import tpu as pltpu
from jax.experimental.pallas import tpu_sc as plsc

B = 16384
K = 8192
D = 32
BETA = 0.25

BM = 256
BK = 2048

NC = 2
NS = 16
NW = NC * NS
CHUNK = B // NW
IDX_ROWS = CHUNK // 128


def _argmin_kernel(a_ref, e_ref, a2_ref, e2_ref, idx_ref):
    a_bf = a_ref[...].astype(jnp.bfloat16)
    a2 = a2_ref[...]
    best_v = jnp.full((BM, 1), jnp.inf, jnp.float32)
    best_i = jnp.zeros((BM, 1), jnp.int32)
    iota = lax.broadcasted_iota(jnp.int32, (BM, BK), 1)
    for k in range(K // BK):
        e_blk = e_ref[pl.ds(k * BK, BK), :]
        e2 = e2_ref[:, pl.ds(k * BK, BK)]
        ae = lax.dot_general(a_bf, e_blk.astype(jnp.bfloat16),
                             (((1,), (1,)), ((), ())),
                             preferred_element_type=jnp.float32)
        dist = (a2 + e2) - 2.0 * ae
        m = jnp.min(dist, axis=1, keepdims=True)
        am = jnp.min(jnp.where(dist == m, iota, K), axis=1, keepdims=True)
        upd = m < best_v
        best_v = jnp.where(upd, m, best_v)
        best_i = jnp.where(upd, am + (k * BK), best_i)
    idx_ref[...] = best_i


def _argmin_call(a, codebook, a2, e2):
    return pl.pallas_call(
        _argmin_kernel,
        grid=(B // BM,),
        in_specs=[
            pl.BlockSpec((BM, D), lambda i: (i, 0)),
            pl.BlockSpec((K, D), lambda i: (0, 0)),
            pl.BlockSpec((BM, 1), lambda i: (i, 0)),
            pl.BlockSpec((1, K), lambda i: (0, 0)),
        ],
        out_specs=pl.BlockSpec((BM, 1), lambda i: (i, 0)),
        out_shape=jax.ShapeDtypeStruct((B, 1), jnp.int32),
    )(a, codebook, a2, e2)


_SC_MESH = plsc.VectorSubcoreMesh(core_axis_name="c", subcore_axis_name="s")


@functools.partial(
    pl.kernel,
    mesh=_SC_MESH,
    compiler_params=pltpu.CompilerParams(use_tc_tiling_on_sc=False),
    out_type=[
        jax.ShapeDtypeStruct((B, D), jnp.float32),
        jax.ShapeDtypeStruct((NC, K), jnp.float32),
        jax.ShapeDtypeStruct((NW, 16), jnp.float32),
    ],
    scratch_types=[
        pltpu.VMEM((IDX_ROWS, 128), jnp.int32),
        pltpu.VMEM((CHUNK, D), jnp.float32),
        pltpu.VMEM((CHUNK, D), jnp.float32),
        pltpu.VMEM((128,), jnp.float32),
        pltpu.VMEM((K,), jnp.float32),
        pltpu.VMEM((16,), jnp.float32),
        pltpu.VMEM_SHARED((K,), jnp.float32),
        pltpu.SemaphoreType.DMA,
    ],
)
def _sc_gather(a_hbm, cb_hbm, idx_hbm, z_hbm, counts_hbm, mse_hbm,
               idx_v, rows_v, az_v, ones_v, zeros_v, acc_v, counts_sh, sem):
    cid = lax.axis_index("c")
    sid = lax.axis_index("s")
    wid = sid * NC + cid
    base = wid * CHUNK

    @pl.when(sid == 0)
    def _():
        def zbody(i, _):
            zeros_v[pl.ds(i * 16, 16)] = jnp.zeros((16,), jnp.float32)
            return 0
        lax.fori_loop(0, K // 16, zbody, 0)
        pltpu.sync_copy(zeros_v, counts_sh)

    pltpu.sync_copy(idx_hbm.at[pl.ds(wid * IDX_ROWS, IDX_ROWS)], idx_v)
    copies = []
    for j in range(IDX_ROWS):
        copies.append(pltpu.async_copy(
            cb_hbm.at[idx_v.at[j]], rows_v.at[pl.ds(j * 128, 128)], sem))
    pltpu.sync_copy(a_hbm.at[pl.ds(base, CHUNK)], az_v)
    for c in copies:
        c.wait()

    def obody(i, _):
        ones_v[pl.ds(i * 16, 16)] = jnp.full((16,), 1.0, jnp.float32)
        return 0
    lax.fori_loop(0, 128 // 16, obody, 0)

    plsc.subcore_barrier()
    for j in range(IDX_ROWS):
        pltpu.sync_copy(ones_v, counts_sh.at[idx_v.at[j]], add=True)

    def rbody(r, acc):
        for h in range(D // 16):
            zr = rows_v[r, pl.ds(h * 16, 16)]
            ar = az_v[r, pl.ds(h * 16, 16)]
            t = zr - ar
            az_v[r, pl.ds(h * 16, 16)] = ar + t
            acc = acc + t * t
        return acc
    acc = lax.fori_loop(0, CHUNK, rbody, jnp.zeros((16,), jnp.float32))
    acc_v[...] = acc
    pltpu.sync_copy(az_v, z_hbm.at[pl.ds(base, CHUNK)])
    pltpu.sync_copy(acc_v, mse_hbm.at[wid])

    plsc.subcore_barrier()

    @pl.when(sid == 0)
    def _():
        pltpu.sync_copy(counts_sh, counts_hbm.at[cid])


def _finalize_kernel(counts_ref, mse_ref, out_ref):
    counts = jnp.sum(counts_ref[...], axis=0, keepdims=True)
    mse = jnp.sum(mse_ref[...]) / float(B * D)
    usage = counts / float(B)
    ent = usage * jnp.log(usage + 1e-10)
    perplexity = jnp.exp(-jnp.sum(ent))
    used = jnp.sum((usage > 0).astype(jnp.float32))
    umean = jnp.sum(usage) / float(K)
    out_ref[0, 0] = BETA * mse
    out_ref[0, 1] = mse
    out_ref[0, 2] = perplexity
    out_ref[0, 3] = used
    out_ref[0, 4] = umean
    out_ref[0, 5] = 0.0
    out_ref[0, 6] = 0.0
    out_ref[0, 7] = 0.0


def _finalize_call(counts2, mse_parts):
    return pl.pallas_call(
        _finalize_kernel,
        in_specs=[
            pl.BlockSpec((NC, K), lambda: (0, 0)),
            pl.BlockSpec((NW, 16), lambda: (0, 0)),
        ],
        out_specs=pl.BlockSpec(memory_space=pltpu.SMEM),
        out_shape=jax.ShapeDtypeStruct((1, 8), jnp.float32),
    )(counts2, mse_parts)


def kernel(a, codebook):
    a2 = jnp.sum(a ** 2, axis=1, keepdims=True)
    e2 = jnp.sum(codebook ** 2, axis=1)[None, :]
    idx2 = _argmin_call(a, codebook, a2, e2)
    idx = idx2.reshape(B)
    idx128 = idx.reshape(NW * IDX_ROWS, 128)
    z, counts2, mse_parts = _sc_gather(a, codebook, idx128)
    scal = _finalize_call(counts2, mse_parts)
    return (idx, z, scal[0, 0], scal[0, 1], scal[0, 2], scal[0, 3],
            scal[0, 4])


# speedup vs baseline: 1.3010x; 1.0759x over previous
import functools

import jax
import jax.numpy as jnp
from jax import lax
from jax.experimental import pallas as pl
from jax.experimental.pallas import tpu as pltpu
from jax.experimental.pallas import tpu_sc as plsc

B = 16384
K = 8192
D = 32
BETA = 0.25

BM = 256
BK = 2048

NC = 2
NS = 16
NW = NC * NS
CHUNK = B // NW
IDX_ROWS = CHUNK // 128


def _argmin_kernel(na_ref, e_ref, e2_ref, idx_ref):
    na_bf = na_ref[...].astype(jnp.bfloat16)
    best_v = jnp.full((BM, 1), jnp.inf, jnp.float32)
    best_i = jnp.zeros((BM, 1), jnp.int32)
    iota = lax.broadcasted_iota(jnp.int32, (BM, BK), 1)
    for k in range(K // BK):
        e_blk = e_ref[pl.ds(k * BK, BK), :]
        e2 = e2_ref[:, pl.ds(k * BK, BK)]
        nae = lax.dot_general(na_bf, e_blk.astype(jnp.bfloat16),
                              (((1,), (1,)), ((), ())),
                              preferred_element_type=jnp.float32)
        dist = e2 + nae
        m = jnp.min(dist, axis=1, keepdims=True)
        am = jnp.min(jnp.where(dist == m, iota, K), axis=1, keepdims=True)
        upd = m < best_v
        best_v = jnp.where(upd, m, best_v)
        best_i = jnp.where(upd, am + (k * BK), best_i)
    idx_ref[...] = best_i


def _argmin_call(na, codebook, e2):
    return pl.pallas_call(
        _argmin_kernel,
        grid=(B // BM,),
        in_specs=[
            pl.BlockSpec((BM, D), lambda i: (i, 0)),
            pl.BlockSpec((K, D), lambda i: (0, 0)),
            pl.BlockSpec((1, K), lambda i: (0, 0)),
        ],
        out_specs=pl.BlockSpec((BM, 1), lambda i: (i, 0)),
        out_shape=jax.ShapeDtypeStruct((B, 1), jnp.int32),
    )(na, codebook, e2)


_SC_MESH = plsc.VectorSubcoreMesh(core_axis_name="c", subcore_axis_name="s")


@functools.partial(
    pl.kernel,
    mesh=_SC_MESH,
    compiler_params=pltpu.CompilerParams(use_tc_tiling_on_sc=False),
    out_type=[
        jax.ShapeDtypeStruct((B, D), jnp.float32),
        jax.ShapeDtypeStruct((NC, K), jnp.float32),
        jax.ShapeDtypeStruct((NW, 16), jnp.float32),
    ],
    scratch_types=[
        pltpu.VMEM((IDX_ROWS, 128), jnp.int32),
        pltpu.VMEM((CHUNK, D), jnp.float32),
        pltpu.VMEM((CHUNK, D), jnp.float32),
        pltpu.VMEM((128,), jnp.float32),
        pltpu.VMEM((K,), jnp.float32),
        pltpu.VMEM((16,), jnp.float32),
        pltpu.VMEM_SHARED((K,), jnp.float32),
        pltpu.SemaphoreType.DMA,
    ],
)
def _sc_gather(a_hbm, cb_hbm, idx_hbm, z_hbm, counts_hbm, mse_hbm,
               idx_v, rows_v, az_v, ones_v, zeros_v, acc_v, counts_sh, sem):
    cid = lax.axis_index("c")
    sid = lax.axis_index("s")
    wid = sid * NC + cid
    base = wid * CHUNK

    @pl.when(sid == 0)
    def _():
        def zbody(i, _):
            zeros_v[pl.ds(i * 16, 16)] = jnp.zeros((16,), jnp.float32)
            return 0
        lax.fori_loop(0, K // 16, zbody, 0)
        pltpu.sync_copy(zeros_v, counts_sh)

    pltpu.sync_copy(idx_hbm.at[pl.ds(wid * IDX_ROWS, IDX_ROWS)], idx_v)
    copies = []
    for j in range(IDX_ROWS):
        copies.append(pltpu.async_copy(
            cb_hbm.at[idx_v.at[j]], rows_v.at[pl.ds(j * 128, 128)], sem))
    pltpu.sync_copy(a_hbm.at[pl.ds(base, CHUNK)], az_v)
    for c in copies:
        c.wait()

    def obody(i, _):
        ones_v[pl.ds(i * 16, 16)] = jnp.full((16,), 1.0, jnp.float32)
        return 0
    lax.fori_loop(0, 128 // 16, obody, 0)

    plsc.subcore_barrier()
    for j in range(IDX_ROWS):
        pltpu.sync_copy(ones_v, counts_sh.at[idx_v.at[j]], add=True)

    def rbody(r, acc):
        for h in range(D // 16):
            zr = rows_v[r, pl.ds(h * 16, 16)]
            ar = az_v[r, pl.ds(h * 16, 16)]
            t = zr - ar
            az_v[r, pl.ds(h * 16, 16)] = ar + t
            acc = acc + t * t
        return acc
    acc = lax.fori_loop(0, CHUNK, rbody, jnp.zeros((16,), jnp.float32))
    acc_v[...] = acc
    pltpu.sync_copy(az_v, z_hbm.at[pl.ds(base, CHUNK)])
    pltpu.sync_copy(acc_v, mse_hbm.at[wid])

    plsc.subcore_barrier()

    @pl.when(sid == 0)
    def _():
        pltpu.sync_copy(counts_sh, counts_hbm.at[cid])


def _finalize_kernel(counts_ref, mse_ref, out_ref):
    counts = jnp.sum(counts_ref[...], axis=0, keepdims=True)
    mse = jnp.sum(mse_ref[...]) / float(B * D)
    usage = counts / float(B)
    ent = usage * jnp.log(usage + 1e-10)
    perplexity = jnp.exp(-jnp.sum(ent))
    used = jnp.sum((usage > 0).astype(jnp.float32))
    umean = jnp.sum(usage) / float(K)
    out_ref[0, 0] = BETA * mse
    out_ref[0, 1] = mse
    out_ref[0, 2] = perplexity
    out_ref[0, 3] = used
    out_ref[0, 4] = umean
    out_ref[0, 5] = 0.0
    out_ref[0, 6] = 0.0
    out_ref[0, 7] = 0.0


def _finalize_call(counts2, mse_parts):
    return pl.pallas_call(
        _finalize_kernel,
        in_specs=[
            pl.BlockSpec((NC, K), lambda: (0, 0)),
            pl.BlockSpec((NW, 16), lambda: (0, 0)),
        ],
        out_specs=pl.BlockSpec(memory_space=pltpu.SMEM),
        out_shape=jax.ShapeDtypeStruct((1, 8), jnp.float32),
    )(counts2, mse_parts)


def kernel(a, codebook):
    e2 = jnp.sum(codebook ** 2, axis=1)[None, :]
    idx2 = _argmin_call(-2.0 * a, codebook, e2)
    idx = idx2.reshape(B)
    idx128 = idx.reshape(NW * IDX_ROWS, 128)
    z, counts2, mse_parts = _sc_gather(a, codebook, idx128)
    scal = _finalize_call(counts2, mse_parts)
    return (idx, z, scal[0, 0], scal[0, 1], scal[0, 2], scal[0, 3],
            scal[0, 4])
